# baseline (speedup 1.0000x reference)
import jax
import jax.numpy as jnp
from jax.experimental import pallas as pl
from jax.experimental.pallas import tpu as pltpu

B = 128
CLS = 8
POOL = 1024
HALF = 6
D = 512
SEQ = 77
HEAD = 1 + 2 * HALF
SUF = SEQ - HEAD
BPS = 8
ROWS = BPS * CLS


def _body(idx_ref, *refs):
    gathers = refs[:4 * BPS]
    (pref, suf, ncpref, ncsuf, gid, aid, tokr, nctokr,
     out_p, out_tok, out_ncp, out_nctok) = refs[4 * BPS:]

    s = pl.program_id(0)
    for m in range(BPS):
        b = s * BPS + m
        g0, g1 = gathers[2 * m], gathers[2 * m + 1]
        a0, a1 = gathers[2 * BPS + 2 * m], gathers[2 * BPS + 2 * m + 1]
        is_g = b < (B // 2)
        r0v = jnp.where(is_g, g0[0], a0[0])
        r1v = jnp.where(is_g, g1[0], a1[0])
        ctx = jnp.concatenate([r0v, r1v], axis=0)
        lo = m * CLS
        out_p[lo:lo + CLS, 0:1, :] = pref[:]
        out_p[lo:lo + CLS, 1:HEAD, :] = jnp.broadcast_to(ctx[None],
                                                         (CLS, 2 * HALF, D))
        out_p[lo:lo + CLS, HEAD:SEQ, :] = suf[:]
        out_tok[lo:lo + CLS, :] = tokr[:]
        out_nctok[lo:lo + CLS, :] = jnp.broadcast_to(nctokr[:], (CLS, SEQ))
    out_ncp[:, 0:1, :] = jnp.broadcast_to(ncpref[:], (ROWS, 1, D))
    out_ncp[:, 1:1 + HALF, :] = gid[:]
    out_ncp[:, 1 + HALF:HEAD, :] = aid[:]
    out_ncp[:, HEAD:SEQ, :] = jnp.broadcast_to(ncsuf[:], (ROWS, SUF, D))


def kernel(indices_g, global_prompt, attribute_prompt, token_prefix,
           token_suffix, nc_token_prefix, nc_token_suffix,
           tokenized_prompts, nc_tokenized_prompts):
    grid = (B // BPS,)

    def gspec(m):
        return pl.BlockSpec(
            (1, HALF, D),
            lambda s, idx, m=m: (idx[(2 * BPS * s + m) % B], 0, 0))

    in_specs = (
        [gspec(m) for m in range(2 * BPS)] +
        [gspec(m) for m in range(2 * BPS)] +
        [
            pl.BlockSpec((CLS, 1, D), lambda s, idx: (0, 0, 0)),
            pl.BlockSpec((CLS, SUF, D), lambda s, idx: (0, 0, 0)),
            pl.BlockSpec((1, 1, D), lambda s, idx: (0, 0, 0)),
            pl.BlockSpec((1, SUF, D), lambda s, idx: (0, 0, 0)),
            pl.BlockSpec((ROWS, HALF, D), lambda s, idx: (s, 0, 0)),
            pl.BlockSpec((ROWS, HALF, D), lambda s, idx: (s, 0, 0)),
            pl.BlockSpec((CLS, SEQ), lambda s, idx: (0, 0)),
            pl.BlockSpec((1, SEQ), lambda s, idx: (0, 0)),
        ])
    out_specs = [
        pl.BlockSpec((ROWS, SEQ, D), lambda s, idx: (s, 0, 0)),
        pl.BlockSpec((ROWS, SEQ), lambda s, idx: (s, 0)),
        pl.BlockSpec((ROWS, SEQ, D), lambda s, idx: (s, 0, 0)),
        pl.BlockSpec((ROWS, SEQ), lambda s, idx: (s, 0)),
    ]
    out_shape = [
        jax.ShapeDtypeStruct((B * CLS, SEQ, D), jnp.float32),
        jax.ShapeDtypeStruct((B * CLS, SEQ), jnp.int32),
        jax.ShapeDtypeStruct((POOL, SEQ, D), jnp.float32),
        jax.ShapeDtypeStruct((POOL, SEQ), jnp.int32),
    ]

    grid_spec = pltpu.PrefetchScalarGridSpec(
        num_scalar_prefetch=1,
        grid=grid,
        in_specs=in_specs,
        out_specs=out_specs,
    )
    prompts, tok, nc_prompts, nc_tok = pl.pallas_call(
        _body,
        grid_spec=grid_spec,
        out_shape=out_shape,
    )(indices_g,
      *([global_prompt] * (2 * BPS)), *([attribute_prompt] * (2 * BPS)),
      token_prefix, token_suffix, nc_token_prefix, nc_token_suffix,
      global_prompt, attribute_prompt, tokenized_prompts,
      nc_tokenized_prompts)

    return (prompts, tok, nc_prompts, nc_tok)
